# Initial kernel scaffold; baseline (speedup 1.0000x reference)
import jax
import jax.numpy as jnp
from jax.experimental import pallas as pl


def kernel(node_x, nodes_map, node2edge_seg, edges_map, edge2node_seg, n2e_Wk0, n2e_Wv0, n2e_q0, n2e_Wo0, e2n_Wk0, e2n_Wv0, e2n_q0, e2n_Wo0, n2e_Wk1, n2e_Wv1, n2e_q1, n2e_Wo1, e2n_Wk1, e2n_Wv1, e2n_q1, e2n_Wo1, out_W1, out_b1, out_W2, out_b2):
    raise NotImplementedError("write your pallas kernel here")


# speedup vs baseline: 36.4362x; 36.4362x over previous
import functools

import jax
import jax.numpy as jnp
from jax import lax
from jax.experimental import pallas as pl
from jax.experimental.pallas import tpu as pltpu
from jax.experimental.pallas import tpu_sc as plsc

_N = 10000
_H = 5000
_M = 320000
_DIM = 64
_C = 80
_NW = 32
_B = 80
_NBLK = 125
_GRP = 5
_NG = _NBLK // _GRP

_HPAD = 5120
_NPAD = 10240


def _prep_body(x_ref, wk_ref, wv_ref, q_ref, o_ref):
    x = x_ref[...]
    k = jnp.dot(x, wk_ref[...], preferred_element_type=jnp.float32)
    v = jnp.dot(x, wv_ref[...], preferred_element_type=jnp.float32)
    s = jnp.sum(k * q_ref[...], axis=1, keepdims=True) * (1.0 / 8.0)
    w = jnp.exp(s - jnp.max(s))
    o_ref[:, :_DIM] = v * w
    o_ref[:, _DIM:] = jnp.broadcast_to(w, (x.shape[0], _C - _DIM))


def _prep(x, wk, wv, q):
    n = x.shape[0]
    return pl.pallas_call(
        _prep_body,
        out_shape=jax.ShapeDtypeStruct((n, _C), jnp.float32),
    )(x, wk, wv, q.reshape(1, _DIM))


def _post_body(p_ref, wo_ref, o_ref):
    t = o_ref.shape[0]
    p = p_ref[0, :t, :] + p_ref[1, :t, :]
    num = p[:, :_DIM]
    den = p[:, _DIM:_DIM + 1]
    y = num / (den + 1e-16)
    o_ref[...] = jnp.maximum(
        jnp.dot(y, wo_ref[...], preferred_element_type=jnp.float32), 0.0)


def _post(partials, wo, t):
    return pl.pallas_call(
        _post_body,
        out_shape=jax.ShapeDtypeStruct((t, _DIM), jnp.float32),
    )(partials, wo)


def _head_body(x_ref, w1_ref, b1_ref, w2_ref, b2_ref, o_ref):
    h = jnp.dot(x_ref[...], w1_ref[...], preferred_element_type=jnp.float32)
    h = jnp.maximum(h + b1_ref[...], 0.0)
    logits = jnp.dot(h, w2_ref[...], preferred_element_type=jnp.float32)
    logits = logits + b2_ref[...]
    m = jnp.max(logits, axis=1, keepdims=True)
    shifted = logits - m
    lse = jnp.log(jnp.sum(jnp.exp(shifted), axis=1, keepdims=True))
    o_ref[...] = shifted - lse


def _head(x, w1, b1, w2, b2):
    n, nc = x.shape[0], w2.shape[1]
    return pl.pallas_call(
        _head_body,
        out_shape=jax.ShapeDtypeStruct((n, nc), jnp.float32),
    )(x, w1, b1.reshape(1, -1), w2, b2.reshape(1, -1))


def _seg_sum_kernel(tpad):
    stripe = tpad // 16
    mesh = plsc.VectorSubcoreMesh(core_axis_name="c", subcore_axis_name="s")

    @functools.partial(
        pl.kernel,
        out_type=jax.ShapeDtypeStruct((2, tpad, _C), jnp.float32),
        mesh=mesh,
        scratch_types=[
            pltpu.VMEM((_NBLK, _B), jnp.int32),
            pltpu.VMEM((_NBLK, _B), jnp.int32),
            pltpu.VMEM((_GRP, _B, _C), jnp.float32),
            pltpu.VMEM_SHARED((tpad, _C), jnp.float32),
            pltpu.SemaphoreType.DMA,
            pltpu.SemaphoreType.DMA,
        ],
        compiler_params=pltpu.CompilerParams(use_tc_tiling_on_sc=False),
    )
    def k(table, map3, seg3, zeros, out, idx_v, seg_v, rows_v, acc,
          sem_g, sem_s):
        c = lax.axis_index("c")
        s = lax.axis_index("s")
        wid = s * 2 + c
        row0 = s * stripe
        pltpu.sync_copy(zeros.at[pl.ds(row0, stripe)],
                        acc.at[pl.ds(row0, stripe)])
        pltpu.sync_copy(map3.at[wid], idx_v)
        pltpu.sync_copy(seg3.at[wid], seg_v)
        plsc.subcore_barrier()

        def body(g, carry):
            gathers = []
            for i in range(_GRP):
                j = g * _GRP + i
                gathers.append(
                    pltpu.async_copy(table.at[idx_v.at[j]], rows_v.at[i],
                                     sem_g))
            for d in gathers:
                d.wait()
            scatters = []
            for i in range(_GRP):
                j = g * _GRP + i
                scatters.append(
                    pltpu.async_copy(rows_v.at[i], acc.at[seg_v.at[j]],
                                     sem_s, add=True))
            for d in scatters:
                d.wait()
            return carry

        lax.fori_loop(0, _NG, body, 0)
        plsc.subcore_barrier()
        pltpu.sync_copy(acc.at[pl.ds(row0, stripe)],
                        out.at[c, pl.ds(row0, stripe)])

    return k


def _seg_sum(table, map3, seg3, tpad):
    zeros = jnp.zeros((tpad, _C), jnp.float32)
    return _seg_sum_kernel(tpad)(table, map3, seg3, zeros)


def kernel(node_x, nodes_map, node2edge_seg, edges_map, edge2node_seg,
           n2e_Wk0, n2e_Wv0, n2e_q0, n2e_Wo0,
           e2n_Wk0, e2n_Wv0, e2n_q0, e2n_Wo0,
           n2e_Wk1, n2e_Wv1, n2e_q1, n2e_Wo1,
           e2n_Wk1, e2n_Wv1, e2n_q1, e2n_Wo1,
           out_W1, out_b1, out_W2, out_b2):
    nm3 = nodes_map.reshape(_NW, _NBLK, _B)
    ns3 = node2edge_seg.reshape(_NW, _NBLK, _B)
    em3 = edges_map.reshape(_NW, _NBLK, _B)
    es3 = edge2node_seg.reshape(_NW, _NBLK, _B)

    n2e = [(n2e_Wk0, n2e_Wv0, n2e_q0, n2e_Wo0),
           (n2e_Wk1, n2e_Wv1, n2e_q1, n2e_Wo1)]
    e2n = [(e2n_Wk0, e2n_Wv0, e2n_q0, e2n_Wo0),
           (e2n_Wk1, e2n_Wv1, e2n_q1, e2n_Wo1)]

    x = node_x
    for i in range(2):
        wk, wv, q, wo = n2e[i]
        table = _prep(x, wk, wv, q)
        part = _seg_sum(table, nm3, ns3, _HPAD)
        edge_x = _post(part, wo, _H)

        wk, wv, q, wo = e2n[i]
        table = _prep(edge_x, wk, wv, q)
        part = _seg_sum(table, em3, es3, _NPAD)
        nx = _post(part, wo, _N)

        x = jnp.concatenate([x, nx], axis=1)

    return _head(x, out_W1, out_b1, out_W2, out_b2)


# speedup vs baseline: 41.3783x; 1.1356x over previous
import functools

import jax
import jax.numpy as jnp
from jax import lax
from jax.experimental import pallas as pl
from jax.experimental.pallas import tpu as pltpu
from jax.experimental.pallas import tpu_sc as plsc

_N = 10000
_H = 5000
_M = 320000
_DIM = 64
_C = 80
_NW = 32
_B = 40
_NBLK = 250
_GRP = 5
_NG = _NBLK // _GRP

_HPAD = 5120
_NPAD = 10240


def _prep_body(x_ref, wk_ref, wv_ref, q_ref, o_ref):
    x = x_ref[...]
    k = jnp.dot(x, wk_ref[...], preferred_element_type=jnp.float32)
    v = jnp.dot(x, wv_ref[...], preferred_element_type=jnp.float32)
    s = jnp.sum(k * q_ref[...], axis=1, keepdims=True) * (1.0 / 8.0)
    w = jnp.exp(s - jnp.max(s))
    o_ref[:, :_DIM] = v * w
    o_ref[:, _DIM:] = jnp.broadcast_to(w, (x.shape[0], _C - _DIM))


def _prep(x, wk, wv, q):
    n = x.shape[0]
    return pl.pallas_call(
        _prep_body,
        out_shape=jax.ShapeDtypeStruct((n, _C), jnp.float32),
    )(x, wk, wv, q.reshape(1, _DIM))


def _post_body(p_ref, wo_ref, o_ref):
    t = o_ref.shape[0]
    p = p_ref[0, :t, :] + p_ref[1, :t, :]
    num = p[:, :_DIM]
    den = p[:, _DIM:_DIM + 1]
    y = num / (den + 1e-16)
    o_ref[...] = jnp.maximum(
        jnp.dot(y, wo_ref[...], preferred_element_type=jnp.float32), 0.0)


def _post(partials, wo, t):
    return pl.pallas_call(
        _post_body,
        out_shape=jax.ShapeDtypeStruct((t, _DIM), jnp.float32),
    )(partials, wo)


def _head_body(x_ref, w1_ref, b1_ref, w2_ref, b2_ref, o_ref):
    h = jnp.dot(x_ref[...], w1_ref[...], preferred_element_type=jnp.float32)
    h = jnp.maximum(h + b1_ref[...], 0.0)
    logits = jnp.dot(h, w2_ref[...], preferred_element_type=jnp.float32)
    logits = logits + b2_ref[...]
    m = jnp.max(logits, axis=1, keepdims=True)
    shifted = logits - m
    lse = jnp.log(jnp.sum(jnp.exp(shifted), axis=1, keepdims=True))
    o_ref[...] = shifted - lse


def _head(x, w1, b1, w2, b2):
    n, nc = x.shape[0], w2.shape[1]
    return pl.pallas_call(
        _head_body,
        out_shape=jax.ShapeDtypeStruct((n, nc), jnp.float32),
    )(x, w1, b1.reshape(1, -1), w2, b2.reshape(1, -1))


def _seg_sum_kernel(tpad):
    stripe = tpad // 16
    mesh = plsc.VectorSubcoreMesh(core_axis_name="c", subcore_axis_name="s")

    @functools.partial(
        pl.kernel,
        out_type=jax.ShapeDtypeStruct((2, tpad, _C), jnp.float32),
        mesh=mesh,
        scratch_types=[
            pltpu.VMEM((_NBLK, _B), jnp.int32),
            pltpu.VMEM((_NBLK, _B), jnp.int32),
            pltpu.VMEM((2, _GRP, _B, _C), jnp.float32),
            pltpu.VMEM_SHARED((tpad, _C), jnp.float32),
            pltpu.SemaphoreType.DMA,
            pltpu.SemaphoreType.DMA,
        ],
        compiler_params=pltpu.CompilerParams(use_tc_tiling_on_sc=False),
    )
    def k(table, map3, seg3, zeros, out, idx_v, seg_v, rows_v, acc,
          sem_g, sem_s):
        c = lax.axis_index("c")
        s = lax.axis_index("s")
        wid = s * 2 + c
        row0 = s * stripe
        pltpu.sync_copy(zeros.at[pl.ds(row0, stripe)],
                        acc.at[pl.ds(row0, stripe)])
        pltpu.sync_copy(map3.at[wid], idx_v)
        pltpu.sync_copy(seg3.at[wid], seg_v)
        plsc.subcore_barrier()

        def fire_gathers(g, bank):
            for i in range(_GRP):
                pltpu.async_copy(table.at[idx_v.at[g * _GRP + i]],
                                 rows_v.at[bank, i], sem_g)

        def drain_gathers(g, bank):
            for i in range(_GRP):
                pltpu.make_async_copy(table.at[idx_v.at[g * _GRP + i]],
                                      rows_v.at[bank, i], sem_g).wait()

        def fire_scatters(g, bank):
            for i in range(_GRP):
                pltpu.async_copy(rows_v.at[bank, i],
                                 acc.at[seg_v.at[g * _GRP + i]],
                                 sem_s, add=True)

        def drain_scatters(g, bank):
            for i in range(_GRP):
                pltpu.make_async_copy(rows_v.at[bank, i],
                                      acc.at[seg_v.at[g * _GRP + i]],
                                      sem_s).wait()

        fire_gathers(0, 0)

        def body(g, carry):
            bank = lax.rem(g, 2)
            drain_gathers(g, bank)
            pl.when(g > 0)(lambda: drain_scatters(g - 1, 1 - bank))
            pl.when(g + 1 < _NG)(lambda: fire_gathers(g + 1, 1 - bank))
            fire_scatters(g, bank)
            return carry

        lax.fori_loop(0, _NG, body, 0)
        drain_scatters(_NG - 1, (_NG - 1) % 2)
        plsc.subcore_barrier()
        pltpu.sync_copy(acc.at[pl.ds(row0, stripe)],
                        out.at[c, pl.ds(row0, stripe)])

    return k


def _seg_sum(table, map3, seg3, tpad):
    zeros = jnp.zeros((tpad, _C), jnp.float32)
    return _seg_sum_kernel(tpad)(table, map3, seg3, zeros)


def kernel(node_x, nodes_map, node2edge_seg, edges_map, edge2node_seg,
           n2e_Wk0, n2e_Wv0, n2e_q0, n2e_Wo0,
           e2n_Wk0, e2n_Wv0, e2n_q0, e2n_Wo0,
           n2e_Wk1, n2e_Wv1, n2e_q1, n2e_Wo1,
           e2n_Wk1, e2n_Wv1, e2n_q1, e2n_Wo1,
           out_W1, out_b1, out_W2, out_b2):
    nm3 = nodes_map.reshape(_NW, _NBLK, _B)
    ns3 = node2edge_seg.reshape(_NW, _NBLK, _B)
    em3 = edges_map.reshape(_NW, _NBLK, _B)
    es3 = edge2node_seg.reshape(_NW, _NBLK, _B)

    n2e = [(n2e_Wk0, n2e_Wv0, n2e_q0, n2e_Wo0),
           (n2e_Wk1, n2e_Wv1, n2e_q1, n2e_Wo1)]
    e2n = [(e2n_Wk0, e2n_Wv0, e2n_q0, e2n_Wo0),
           (e2n_Wk1, e2n_Wv1, e2n_q1, e2n_Wo1)]

    x = node_x
    for i in range(2):
        wk, wv, q, wo = n2e[i]
        table = _prep(x, wk, wv, q)
        part = _seg_sum(table, nm3, ns3, _HPAD)
        edge_x = _post(part, wo, _H)

        wk, wv, q, wo = e2n[i]
        table = _prep(edge_x, wk, wv, q)
        part = _seg_sum(table, em3, es3, _NPAD)
        nx = _post(part, wo, _N)

        x = jnp.concatenate([x, nx], axis=1)

    return _head(x, out_W1, out_b1, out_W2, out_b2)


# speedup vs baseline: 42.2570x; 1.0212x over previous
import functools

import jax
import jax.numpy as jnp
from jax import lax
from jax.experimental import pallas as pl
from jax.experimental.pallas import tpu as pltpu
from jax.experimental.pallas import tpu_sc as plsc

_N = 10000
_H = 5000
_M = 320000
_DIM = 64
_C = 80
_NW = 32
_B = 125
_NBLK = 80
_GRP = 2
_NG = _NBLK // _GRP

_HPAD = 5120
_NPAD = 10240


def _prep_body(x_ref, wk_ref, wv_ref, q_ref, o_ref):
    x = x_ref[...]
    k = jnp.dot(x, wk_ref[...], preferred_element_type=jnp.float32)
    v = jnp.dot(x, wv_ref[...], preferred_element_type=jnp.float32)
    s = jnp.sum(k * q_ref[...], axis=1, keepdims=True) * (1.0 / 8.0)
    w = jnp.exp(s - jnp.max(s))
    o_ref[:, :_DIM] = v * w
    o_ref[:, _DIM:] = jnp.broadcast_to(w, (x.shape[0], _C - _DIM))


def _prep(x, wk, wv, q):
    n = x.shape[0]
    return pl.pallas_call(
        _prep_body,
        out_shape=jax.ShapeDtypeStruct((n, _C), jnp.float32),
    )(x, wk, wv, q.reshape(1, _DIM))


def _post_body(p_ref, wo_ref, o_ref):
    t = o_ref.shape[0]
    p = p_ref[0, :t, :] + p_ref[1, :t, :]
    num = p[:, :_DIM]
    den = p[:, _DIM:_DIM + 1]
    y = num / (den + 1e-16)
    o_ref[...] = jnp.maximum(
        jnp.dot(y, wo_ref[...], preferred_element_type=jnp.float32), 0.0)


def _post(partials, wo, t):
    return pl.pallas_call(
        _post_body,
        out_shape=jax.ShapeDtypeStruct((t, _DIM), jnp.float32),
    )(partials, wo)


def _head_body(x_ref, w1_ref, b1_ref, w2_ref, b2_ref, o_ref):
    h = jnp.dot(x_ref[...], w1_ref[...], preferred_element_type=jnp.float32)
    h = jnp.maximum(h + b1_ref[...], 0.0)
    logits = jnp.dot(h, w2_ref[...], preferred_element_type=jnp.float32)
    logits = logits + b2_ref[...]
    m = jnp.max(logits, axis=1, keepdims=True)
    shifted = logits - m
    lse = jnp.log(jnp.sum(jnp.exp(shifted), axis=1, keepdims=True))
    o_ref[...] = shifted - lse


def _head(x, w1, b1, w2, b2):
    n, nc = x.shape[0], w2.shape[1]
    return pl.pallas_call(
        _head_body,
        out_shape=jax.ShapeDtypeStruct((n, nc), jnp.float32),
    )(x, w1, b1.reshape(1, -1), w2, b2.reshape(1, -1))


def _seg_sum_kernel(tpad):
    stripe = tpad // 16
    mesh = plsc.VectorSubcoreMesh(core_axis_name="c", subcore_axis_name="s")

    @functools.partial(
        pl.kernel,
        out_type=jax.ShapeDtypeStruct((2, tpad, _C), jnp.float32),
        mesh=mesh,
        scratch_types=[
            pltpu.VMEM((_NBLK, _B), jnp.int32),
            pltpu.VMEM((_NBLK, _B), jnp.int32),
            pltpu.VMEM((2, _GRP, _B, _C), jnp.float32),
            pltpu.VMEM_SHARED((tpad, _C), jnp.float32),
            pltpu.SemaphoreType.DMA,
            pltpu.SemaphoreType.DMA,
        ],
        compiler_params=pltpu.CompilerParams(use_tc_tiling_on_sc=False),
    )
    def k(table, map3, seg3, zeros, out, idx_v, seg_v, rows_v, acc,
          sem_g, sem_s):
        c = lax.axis_index("c")
        s = lax.axis_index("s")
        wid = s * 2 + c
        row0 = s * stripe
        pltpu.sync_copy(zeros.at[pl.ds(row0, stripe)],
                        acc.at[pl.ds(row0, stripe)])
        pltpu.sync_copy(map3.at[wid], idx_v)
        pltpu.sync_copy(seg3.at[wid], seg_v)
        plsc.subcore_barrier()

        def fire_gathers(g, bank):
            for i in range(_GRP):
                pltpu.async_copy(table.at[idx_v.at[g * _GRP + i]],
                                 rows_v.at[bank, i], sem_g)

        def drain_gathers(g, bank):
            for i in range(_GRP):
                pltpu.make_async_copy(table.at[idx_v.at[g * _GRP + i]],
                                      rows_v.at[bank, i], sem_g).wait()

        def fire_scatters(g, bank):
            for i in range(_GRP):
                pltpu.async_copy(rows_v.at[bank, i],
                                 acc.at[seg_v.at[g * _GRP + i]],
                                 sem_s, add=True)

        def drain_scatters(g, bank):
            for i in range(_GRP):
                pltpu.make_async_copy(rows_v.at[bank, i],
                                      acc.at[seg_v.at[g * _GRP + i]],
                                      sem_s).wait()

        fire_gathers(0, 0)

        def body(g, carry):
            bank = lax.rem(g, 2)
            drain_gathers(g, bank)
            pl.when(g > 0)(lambda: drain_scatters(g - 1, 1 - bank))
            pl.when(g + 1 < _NG)(lambda: fire_gathers(g + 1, 1 - bank))
            fire_scatters(g, bank)
            return carry

        lax.fori_loop(0, _NG, body, 0)
        drain_scatters(_NG - 1, (_NG - 1) % 2)
        plsc.subcore_barrier()
        pltpu.sync_copy(acc.at[pl.ds(row0, stripe)],
                        out.at[c, pl.ds(row0, stripe)])

    return k


def _seg_sum(table, map3, seg3, tpad):
    zeros = jnp.zeros((tpad, _C), jnp.float32)
    return _seg_sum_kernel(tpad)(table, map3, seg3, zeros)


def kernel(node_x, nodes_map, node2edge_seg, edges_map, edge2node_seg,
           n2e_Wk0, n2e_Wv0, n2e_q0, n2e_Wo0,
           e2n_Wk0, e2n_Wv0, e2n_q0, e2n_Wo0,
           n2e_Wk1, n2e_Wv1, n2e_q1, n2e_Wo1,
           e2n_Wk1, e2n_Wv1, e2n_q1, e2n_Wo1,
           out_W1, out_b1, out_W2, out_b2):
    nm3 = nodes_map.reshape(_NW, _NBLK, _B)
    ns3 = node2edge_seg.reshape(_NW, _NBLK, _B)
    em3 = edges_map.reshape(_NW, _NBLK, _B)
    es3 = edge2node_seg.reshape(_NW, _NBLK, _B)

    n2e = [(n2e_Wk0, n2e_Wv0, n2e_q0, n2e_Wo0),
           (n2e_Wk1, n2e_Wv1, n2e_q1, n2e_Wo1)]
    e2n = [(e2n_Wk0, e2n_Wv0, e2n_q0, e2n_Wo0),
           (e2n_Wk1, e2n_Wv1, e2n_q1, e2n_Wo1)]

    x = node_x
    for i in range(2):
        wk, wv, q, wo = n2e[i]
        table = _prep(x, wk, wv, q)
        part = _seg_sum(table, nm3, ns3, _HPAD)
        edge_x = _post(part, wo, _H)

        wk, wv, q, wo = e2n[i]
        table = _prep(edge_x, wk, wv, q)
        part = _seg_sum(table, em3, es3, _NPAD)
        nx = _post(part, wo, _N)

        x = jnp.concatenate([x, nx], axis=1)

    return _head(x, out_W1, out_b1, out_W2, out_b2)


# speedup vs baseline: 45.1788x; 1.0691x over previous
import functools

import jax
import jax.numpy as jnp
from jax import lax
from jax.experimental import pallas as pl
from jax.experimental.pallas import tpu as pltpu
from jax.experimental.pallas import tpu_sc as plsc

_N = 10000
_H = 5000
_M = 320000
_DIM = 64
_C = 80
_NW = 32
_B = 125
_NBLK = 80
_GRP = 2
_NG = _NBLK // _GRP

_HPAD = 5120
_NPAD = 10240


def _prep_math(x, wk, wv, q):
    k = jnp.dot(x, wk, preferred_element_type=jnp.float32)
    v = jnp.dot(x, wv, preferred_element_type=jnp.float32)
    s = jnp.sum(k * q, axis=1, keepdims=True) * (1.0 / 8.0)
    w = jnp.exp(s - jnp.max(s))
    return jnp.concatenate(
        [v * w, jnp.broadcast_to(w, (x.shape[0], _C - _DIM))], axis=1)


def _post_math(p_ref, wo, t):
    p = p_ref[0, :t, :] + p_ref[1, :t, :]
    num = p[:, :_DIM]
    den = p[:, _DIM:_DIM + 1]
    return jnp.maximum(
        jnp.dot(num / (den + 1e-16), wo, preferred_element_type=jnp.float32),
        0.0)


def _prep_body(x_ref, wk_ref, wv_ref, q_ref, o_ref):
    o_ref[...] = _prep_math(x_ref[...], wk_ref[...], wv_ref[...], q_ref[...])


def _prep(x, wk, wv, q):
    n = x.shape[0]
    return pl.pallas_call(
        _prep_body,
        out_shape=jax.ShapeDtypeStruct((n, _C), jnp.float32),
    )(x, wk, wv, q.reshape(1, _DIM))


def _post_prep_body(p_ref, wo_ref, wk_ref, wv_ref, q_ref, o_ref):
    y = _post_math(p_ref, wo_ref[...], o_ref.shape[0])
    o_ref[...] = _prep_math(y, wk_ref[...], wv_ref[...], q_ref[...])


def _post_prep(partials, wo, wk, wv, q, t):
    return pl.pallas_call(
        _post_prep_body,
        out_shape=jax.ShapeDtypeStruct((t, _C), jnp.float32),
    )(partials, wo, wk, wv, q.reshape(1, _DIM))


def _post_cat_prep_body(p_ref, wo_ref, x_ref, wk_ref, wv_ref, q_ref,
                        tab_ref, x2_ref):
    y = _post_math(p_ref, wo_ref[...], x_ref.shape[0])
    x2 = jnp.concatenate([x_ref[...], y], axis=1)
    x2_ref[...] = x2
    tab_ref[...] = _prep_math(x2, wk_ref[...], wv_ref[...], q_ref[...])


def _post_cat_prep(partials, wo, x, wk, wv, q):
    n, d = x.shape
    return pl.pallas_call(
        _post_cat_prep_body,
        out_shape=[jax.ShapeDtypeStruct((n, _C), jnp.float32),
                   jax.ShapeDtypeStruct((n, d + _DIM), jnp.float32)],
    )(partials, wo, x, wk, wv, q.reshape(1, _DIM))


def _post_cat_head_body(p_ref, wo_ref, x_ref, w1_ref, b1_ref, w2_ref, b2_ref,
                        o_ref):
    y = _post_math(p_ref, wo_ref[...], x_ref.shape[0])
    x2 = jnp.concatenate([x_ref[...], y], axis=1)
    h = jnp.dot(x2, w1_ref[...], preferred_element_type=jnp.float32)
    h = jnp.maximum(h + b1_ref[...], 0.0)
    logits = jnp.dot(h, w2_ref[...], preferred_element_type=jnp.float32)
    logits = logits + b2_ref[...]
    m = jnp.max(logits, axis=1, keepdims=True)
    shifted = logits - m
    lse = jnp.log(jnp.sum(jnp.exp(shifted), axis=1, keepdims=True))
    o_ref[...] = shifted - lse


def _post_cat_head(partials, wo, x, w1, b1, w2, b2):
    n, nc = x.shape[0], w2.shape[1]
    return pl.pallas_call(
        _post_cat_head_body,
        out_shape=jax.ShapeDtypeStruct((n, nc), jnp.float32),
    )(partials, wo, x, w1, b1.reshape(1, -1), w2, b2.reshape(1, -1))


def _seg_sum_kernel(tpad):
    stripe = tpad // 16
    mesh = plsc.VectorSubcoreMesh(core_axis_name="c", subcore_axis_name="s")

    @functools.partial(
        pl.kernel,
        out_type=jax.ShapeDtypeStruct((2, tpad, _C), jnp.float32),
        mesh=mesh,
        scratch_types=[
            pltpu.VMEM((_NBLK, _B), jnp.int32),
            pltpu.VMEM((_NBLK, _B), jnp.int32),
            pltpu.VMEM((2, _GRP, _B, _C), jnp.float32),
            pltpu.VMEM_SHARED((tpad, _C), jnp.float32),
            pltpu.SemaphoreType.DMA,
            pltpu.SemaphoreType.DMA,
        ],
        compiler_params=pltpu.CompilerParams(use_tc_tiling_on_sc=False),
    )
    def k(table, map3, seg3, zeros, out, idx_v, seg_v, rows_v, acc,
          sem_g, sem_s):
        c = lax.axis_index("c")
        s = lax.axis_index("s")
        wid = s * 2 + c
        row0 = s * stripe
        pltpu.sync_copy(zeros.at[pl.ds(row0, stripe)],
                        acc.at[pl.ds(row0, stripe)])
        pltpu.sync_copy(map3.at[wid], idx_v)
        pltpu.sync_copy(seg3.at[wid], seg_v)
        plsc.subcore_barrier()

        def fire_gathers(g, bank):
            for i in range(_GRP):
                pltpu.async_copy(table.at[idx_v.at[g * _GRP + i]],
                                 rows_v.at[bank, i], sem_g)

        def drain_gathers(g, bank):
            for i in range(_GRP):
                pltpu.make_async_copy(table.at[idx_v.at[g * _GRP + i]],
                                      rows_v.at[bank, i], sem_g).wait()

        def fire_scatters(g, bank):
            for i in range(_GRP):
                pltpu.async_copy(rows_v.at[bank, i],
                                 acc.at[seg_v.at[g * _GRP + i]],
                                 sem_s, add=True)

        def drain_scatters(g, bank):
            for i in range(_GRP):
                pltpu.make_async_copy(rows_v.at[bank, i],
                                      acc.at[seg_v.at[g * _GRP + i]],
                                      sem_s).wait()

        fire_gathers(0, 0)

        def body(g, carry):
            bank = lax.rem(g, 2)
            drain_gathers(g, bank)
            pl.when(g > 0)(lambda: drain_scatters(g - 1, 1 - bank))
            pl.when(g + 1 < _NG)(lambda: fire_gathers(g + 1, 1 - bank))
            fire_scatters(g, bank)
            return carry

        lax.fori_loop(0, _NG, body, 0)
        drain_scatters(_NG - 1, (_NG - 1) % 2)
        plsc.subcore_barrier()
        pltpu.sync_copy(acc.at[pl.ds(row0, stripe)],
                        out.at[c, pl.ds(row0, stripe)])

    return k


def _seg_sum(table, map3, seg3, tpad):
    zeros = jnp.zeros((tpad, _C), jnp.float32)
    return _seg_sum_kernel(tpad)(table, map3, seg3, zeros)


def kernel(node_x, nodes_map, node2edge_seg, edges_map, edge2node_seg,
           n2e_Wk0, n2e_Wv0, n2e_q0, n2e_Wo0,
           e2n_Wk0, e2n_Wv0, e2n_q0, e2n_Wo0,
           n2e_Wk1, n2e_Wv1, n2e_q1, n2e_Wo1,
           e2n_Wk1, e2n_Wv1, e2n_q1, e2n_Wo1,
           out_W1, out_b1, out_W2, out_b2):
    nm3 = nodes_map.reshape(_NW, _NBLK, _B)
    ns3 = node2edge_seg.reshape(_NW, _NBLK, _B)
    em3 = edges_map.reshape(_NW, _NBLK, _B)
    es3 = edge2node_seg.reshape(_NW, _NBLK, _B)

    n2e = [(n2e_Wk0, n2e_Wv0, n2e_q0, n2e_Wo0),
           (n2e_Wk1, n2e_Wv1, n2e_q1, n2e_Wo1)]
    e2n = [(e2n_Wk0, e2n_Wv0, e2n_q0, e2n_Wo0),
           (e2n_Wk1, e2n_Wv1, e2n_q1, e2n_Wo1)]

    x = node_x
    table = _prep(x, *n2e[0][:3])
    for i in range(2):
        part = _seg_sum(table, nm3, ns3, _HPAD)
        table = _post_prep(part, n2e[i][3], *e2n[i][:3], _H)
        part = _seg_sum(table, em3, es3, _NPAD)
        if i == 0:
            table, x = _post_cat_prep(part, e2n[i][3], x, *n2e[i + 1][:3])
        else:
            return _post_cat_head(part, e2n[i][3], x,
                                  out_W1, out_b1, out_W2, out_b2)


# speedup vs baseline: 45.9398x; 1.0168x over previous
import functools

import jax
import jax.numpy as jnp
from jax import lax
from jax.experimental import pallas as pl
from jax.experimental.pallas import tpu as pltpu
from jax.experimental.pallas import tpu_sc as plsc

_N = 10000
_H = 5000
_M = 320000
_DIM = 64
_C = 80
_NW = 32
_B = 125
_NBLK = 80
_GRP = 2
_NG = _NBLK // _GRP

_HPAD = 5120
_NPAD = 10240


def _prep_math(x, wk, wv, q):
    k = jnp.dot(x, wk, preferred_element_type=jnp.float32)
    v = jnp.dot(x, wv, preferred_element_type=jnp.float32)
    s = jnp.sum(k * q, axis=1, keepdims=True) * (1.0 / 8.0)
    w = jnp.exp(s - jnp.max(s))
    return jnp.concatenate(
        [v * w, jnp.broadcast_to(w, (x.shape[0], _C - _DIM))], axis=1)


def _post_math(p_ref, wo, t):
    p = p_ref[0, :t, :] + p_ref[1, :t, :]
    num = p[:, :_DIM]
    den = p[:, _DIM:_DIM + 1]
    return jnp.maximum(
        jnp.dot(num / (den + 1e-16), wo, preferred_element_type=jnp.float32),
        0.0)


def _prep_body(x_ref, wk_ref, wv_ref, q_ref, o_ref):
    o_ref[...] = _prep_math(x_ref[...], wk_ref[...], wv_ref[...], q_ref[...])


def _prep(x, wk, wv, q):
    n = x.shape[0]
    return pl.pallas_call(
        _prep_body,
        out_shape=jax.ShapeDtypeStruct((n, _C), jnp.float32),
    )(x, wk, wv, q.reshape(1, _DIM))


def _post_prep_body(p_ref, wo_ref, wk_ref, wv_ref, q_ref, o_ref):
    y = _post_math(p_ref, wo_ref[...], o_ref.shape[0])
    o_ref[...] = _prep_math(y, wk_ref[...], wv_ref[...], q_ref[...])


def _post_prep(partials, wo, wk, wv, q, t):
    return pl.pallas_call(
        _post_prep_body,
        out_shape=jax.ShapeDtypeStruct((t, _C), jnp.float32),
    )(partials, wo, wk, wv, q.reshape(1, _DIM))


def _post_cat_prep_body(p_ref, wo_ref, x_ref, wk_ref, wv_ref, q_ref,
                        tab_ref, x2_ref):
    y = _post_math(p_ref, wo_ref[...], x_ref.shape[0])
    x2 = jnp.concatenate([x_ref[...], y], axis=1)
    x2_ref[...] = x2
    tab_ref[...] = _prep_math(x2, wk_ref[...], wv_ref[...], q_ref[...])


def _post_cat_prep(partials, wo, x, wk, wv, q):
    n, d = x.shape
    return pl.pallas_call(
        _post_cat_prep_body,
        out_shape=[jax.ShapeDtypeStruct((n, _C), jnp.float32),
                   jax.ShapeDtypeStruct((n, d + _DIM), jnp.float32)],
    )(partials, wo, x, wk, wv, q.reshape(1, _DIM))


def _post_cat_head_body(p_ref, wo_ref, x_ref, w1_ref, b1_ref, w2_ref, b2_ref,
                        o_ref):
    y = _post_math(p_ref, wo_ref[...], x_ref.shape[0])
    x2 = jnp.concatenate([x_ref[...], y], axis=1)
    h = jnp.dot(x2, w1_ref[...], preferred_element_type=jnp.float32)
    h = jnp.maximum(h + b1_ref[...], 0.0)
    logits = jnp.dot(h, w2_ref[...], preferred_element_type=jnp.float32)
    logits = logits + b2_ref[...]
    m = jnp.max(logits, axis=1, keepdims=True)
    shifted = logits - m
    lse = jnp.log(jnp.sum(jnp.exp(shifted), axis=1, keepdims=True))
    o_ref[...] = shifted - lse


def _post_cat_head(partials, wo, x, w1, b1, w2, b2):
    n, nc = x.shape[0], w2.shape[1]
    return pl.pallas_call(
        _post_cat_head_body,
        out_shape=jax.ShapeDtypeStruct((n, nc), jnp.float32),
    )(partials, wo, x, w1, b1.reshape(1, -1), w2, b2.reshape(1, -1))


def _seg_sum_kernel(tpad):
    stripe = tpad // 16
    mesh = plsc.VectorSubcoreMesh(core_axis_name="c", subcore_axis_name="s")

    @functools.partial(
        pl.kernel,
        out_type=jax.ShapeDtypeStruct((2, tpad, _C), jnp.float32),
        mesh=mesh,
        scratch_types=[
            pltpu.VMEM((_NBLK, _B), jnp.int32),
            pltpu.VMEM((_NBLK, _B), jnp.int32),
            pltpu.VMEM((2, _GRP, _B, _C), jnp.float32),
            pltpu.VMEM_SHARED((tpad, _C), jnp.float32),
            pltpu.SemaphoreType.DMA,
            pltpu.SemaphoreType.DMA,
        ],
        compiler_params=pltpu.CompilerParams(use_tc_tiling_on_sc=False),
    )
    def k(table, map3, seg3, zeros, out, idx_v, seg_v, rows_v, acc,
          sem_g, sem_s):
        c = lax.axis_index("c")
        s = lax.axis_index("s")
        wid = s * 2 + c
        row0 = s * stripe
        d0 = pltpu.async_copy(zeros.at[pl.ds(row0, stripe)],
                              acc.at[pl.ds(row0, stripe)], sem_g)
        d1 = pltpu.async_copy(map3.at[wid], idx_v, sem_g)
        d2 = pltpu.async_copy(seg3.at[wid], seg_v, sem_g)
        d0.wait()
        d1.wait()
        d2.wait()
        plsc.subcore_barrier()

        def fire_gathers(g, bank):
            for i in range(_GRP):
                pltpu.async_copy(table.at[idx_v.at[g * _GRP + i]],
                                 rows_v.at[bank, i], sem_g)

        def drain_gathers(g, bank):
            for i in range(_GRP):
                pltpu.make_async_copy(table.at[idx_v.at[g * _GRP + i]],
                                      rows_v.at[bank, i], sem_g).wait()

        def fire_scatters(g, bank):
            for i in range(_GRP):
                pltpu.async_copy(rows_v.at[bank, i],
                                 acc.at[seg_v.at[g * _GRP + i]],
                                 sem_s, add=True)

        def drain_scatters(g, bank):
            for i in range(_GRP):
                pltpu.make_async_copy(rows_v.at[bank, i],
                                      acc.at[seg_v.at[g * _GRP + i]],
                                      sem_s).wait()

        fire_gathers(0, 0)

        def body(g, carry):
            bank = lax.rem(g, 2)
            drain_gathers(g, bank)
            pl.when(g > 0)(lambda: drain_scatters(g - 1, 1 - bank))
            pl.when(g + 1 < _NG)(lambda: fire_gathers(g + 1, 1 - bank))
            fire_scatters(g, bank)
            return carry

        lax.fori_loop(0, _NG, body, 0)
        drain_scatters(_NG - 1, (_NG - 1) % 2)
        plsc.subcore_barrier()
        pltpu.sync_copy(acc.at[pl.ds(row0, stripe)],
                        out.at[c, pl.ds(row0, stripe)])

    return k


def _seg_sum(table, map3, seg3, tpad):
    zeros = jnp.zeros((tpad, _C), jnp.float32)
    return _seg_sum_kernel(tpad)(table, map3, seg3, zeros)


def kernel(node_x, nodes_map, node2edge_seg, edges_map, edge2node_seg,
           n2e_Wk0, n2e_Wv0, n2e_q0, n2e_Wo0,
           e2n_Wk0, e2n_Wv0, e2n_q0, e2n_Wo0,
           n2e_Wk1, n2e_Wv1, n2e_q1, n2e_Wo1,
           e2n_Wk1, e2n_Wv1, e2n_q1, e2n_Wo1,
           out_W1, out_b1, out_W2, out_b2):
    nm3 = nodes_map.reshape(_NW, _NBLK, _B)
    ns3 = node2edge_seg.reshape(_NW, _NBLK, _B)
    em3 = edges_map.reshape(_NW, _NBLK, _B)
    es3 = edge2node_seg.reshape(_NW, _NBLK, _B)

    n2e = [(n2e_Wk0, n2e_Wv0, n2e_q0, n2e_Wo0),
           (n2e_Wk1, n2e_Wv1, n2e_q1, n2e_Wo1)]
    e2n = [(e2n_Wk0, e2n_Wv0, e2n_q0, e2n_Wo0),
           (e2n_Wk1, e2n_Wv1, e2n_q1, e2n_Wo1)]

    x = node_x
    table = _prep(x, *n2e[0][:3])
    for i in range(2):
        part = _seg_sum(table, nm3, ns3, _HPAD)
        table = _post_prep(part, n2e[i][3], *e2n[i][:3], _H)
        part = _seg_sum(table, em3, es3, _NPAD)
        if i == 0:
            table, x = _post_cat_prep(part, e2n[i][3], x, *n2e[i + 1][:3])
        else:
            return _post_cat_head(part, e2n[i][3], x,
                                  out_W1, out_b1, out_W2, out_b2)
